# baseline (speedup 1.0000x reference)
import jax
import jax.numpy as jnp
from jax import lax
from jax.experimental import pallas as pl
from jax.experimental.pallas import tpu as pltpu
from jax.experimental.pallas import tpu_sc as plsc

_EPS = 1e-05
_BLK = 2048
_E = 16
_NW = 32
_L = 16


def _layernorm_logits(x, w):
    h = x.shape[1]
    s1 = jnp.sum(x, axis=1, keepdims=True)
    s2 = jnp.sum(x * x, axis=1, keepdims=True)
    mu = s1 / h
    var = s2 / h - mu * mu
    rstd = lax.rsqrt(var + 1e-5)
    hn = (x - mu) * rstd
    g = lax.dot_general(
        hn, w, (((1,), (1,)), ((), ())),
        preferred_element_type=jnp.float32)
    return jnp.clip(g, -20.0, 20.0)


def _logits_block(x_ref, w_ref, logits_ref):
    logits_ref[...] = _layernorm_logits(x_ref[...], w_ref[...])


def _router_block(x_ref, w_ref, p_ref, i_ref, logits_ref):
    logits = _layernorm_logits(x_ref[...], w_ref[...])
    logits_ref[...] = logits
    bb, ee = logits.shape
    m = jnp.max(logits, axis=1, keepdims=True)
    ex = jnp.exp(logits - m)
    p = ex / jnp.sum(ex, axis=1, keepdims=True)
    p = jnp.clip(p, _EPS, 1.0)
    iota = lax.broadcasted_iota(jnp.int32, (bb, ee), 1)
    m1 = jnp.max(p, axis=1, keepdims=True)
    i1 = jnp.min(jnp.where(p == m1, iota, ee), axis=1, keepdims=True)
    masked = jnp.where(iota == i1, -1.0, p)
    m2 = jnp.max(masked, axis=1, keepdims=True)
    i2 = jnp.min(jnp.where(masked == m2, iota, ee), axis=1, keepdims=True)
    ps = jnp.maximum(m1 + m2, _EPS)
    p_ref[:, 0:1] = m1 / ps
    p_ref[:, 1:2] = m2 / ps
    i_ref[:, 0:1] = i1
    i_ref[:, 1:2] = i2


def _route_sc(logits_hbm, p1_hbm, p2_hbm, i1_hbm, i2_hbm,
              chunk_v, p1_v, p2_v, i1_v, i2_v):
    c = p1_v.shape[0]
    wid = lax.axis_index("s") * 2 + lax.axis_index("c")
    base = wid * c
    pltpu.sync_copy(logits_hbm.at[pl.ds(base, c), :], chunk_v)
    lane = lax.iota(jnp.int32, _L)
    for g in range(c // _L):
        row = lane + g * _L
        cols = [plsc.load_gather(chunk_v, [row, jnp.full((_L,), e, jnp.int32)])
                for e in range(_E)]
        m = cols[0]
        for e in range(1, _E):
            m = jnp.maximum(m, cols[e])
        exps = [jnp.exp(x - m) for x in cols]
        s = exps[0]
        for e in range(1, _E):
            s = s + exps[e]
        rinv = 1.0 / s
        v1 = jnp.clip(exps[0] * rinv, _EPS, 1.0)
        i1 = jnp.zeros((_L,), jnp.int32)
        v2 = jnp.full((_L,), -1.0, jnp.float32)
        i2 = jnp.zeros((_L,), jnp.int32)
        for e in range(1, _E):
            p = jnp.clip(exps[e] * rinv, _EPS, 1.0)
            gt1 = p > v1
            gt2 = p > v2
            v2 = jnp.where(gt1, v1, jnp.where(gt2, p, v2))
            i2 = jnp.where(gt1, i1, jnp.where(gt2, jnp.full((_L,), e, jnp.int32), i2))
            v1 = jnp.where(gt1, p, v1)
            i1 = jnp.where(gt1, jnp.full((_L,), e, jnp.int32), i1)
        ps = jnp.maximum(v1 + v2, _EPS)
        rs = 1.0 / ps
        sl = pl.ds(g * _L, _L)
        p1_v[sl] = v1 * rs
        p2_v[sl] = v2 * rs
        i1_v[sl] = i1
        i2_v[sl] = i2
    out_sl = pl.ds(base, c)
    pltpu.sync_copy(p1_v, p1_hbm.at[out_sl])
    pltpu.sync_copy(p2_v, p2_hbm.at[out_sl])
    pltpu.sync_copy(i1_v, i1_hbm.at[out_sl])
    pltpu.sync_copy(i2_v, i2_hbm.at[out_sl])


def kernel(hidden_states, gate_weight):
    b, s, h = hidden_states.shape
    e = gate_weight.shape[0]
    n = b * s
    na = n // 2
    nb = n - na
    c = na // _NW
    x = hidden_states.reshape(n, h)

    logits_a = pl.pallas_call(
        _logits_block,
        grid=(na // _BLK,),
        in_specs=[
            pl.BlockSpec((_BLK, h), lambda i: (i, 0)),
            pl.BlockSpec((e, h), lambda i: (0, 0)),
        ],
        out_specs=pl.BlockSpec((_BLK, e), lambda i: (i, 0)),
        out_shape=jax.ShapeDtypeStruct((na, e), jnp.float32),
        compiler_params=pltpu.CompilerParams(
            dimension_semantics=("arbitrary",)),
    )(x[:na], gate_weight)

    route = pl.kernel(
        _route_sc,
        out_type=[
            jax.ShapeDtypeStruct((na,), jnp.float32),
            jax.ShapeDtypeStruct((na,), jnp.float32),
            jax.ShapeDtypeStruct((na,), jnp.int32),
            jax.ShapeDtypeStruct((na,), jnp.int32),
        ],
        mesh=plsc.VectorSubcoreMesh(core_axis_name="c", subcore_axis_name="s"),
        compiler_params=pltpu.CompilerParams(needs_layout_passes=False),
        scratch_types=[
            pltpu.VMEM((c, _E), jnp.float32),
            pltpu.VMEM((c,), jnp.float32),
            pltpu.VMEM((c,), jnp.float32),
            pltpu.VMEM((c,), jnp.int32),
            pltpu.VMEM((c,), jnp.int32),
        ],
    )
    pa1, pa2, ia1, ia2 = route(logits_a)

    pb, ib, logits_b = pl.pallas_call(
        _router_block,
        grid=(nb // _BLK,),
        in_specs=[
            pl.BlockSpec((_BLK, h), lambda i: (i, 0)),
            pl.BlockSpec((e, h), lambda i: (0, 0)),
        ],
        out_specs=[
            pl.BlockSpec((_BLK, 2), lambda i: (i, 0)),
            pl.BlockSpec((_BLK, 2), lambda i: (i, 0)),
            pl.BlockSpec((_BLK, e), lambda i: (i, 0)),
        ],
        out_shape=[
            jax.ShapeDtypeStruct((nb, 2), jnp.float32),
            jax.ShapeDtypeStruct((nb, 2), jnp.int32),
            jax.ShapeDtypeStruct((nb, e), jnp.float32),
        ],
        compiler_params=pltpu.CompilerParams(
            dimension_semantics=("arbitrary",)),
    )(x[na:], gate_weight)

    top_k_probs = jnp.concatenate(
        [jnp.stack([pa1, pa2], axis=-1), pb], axis=0)
    top_k_indices = jnp.concatenate(
        [jnp.stack([ia1, ia2], axis=-1), ib], axis=0)
    router_logits = jnp.concatenate([logits_a, logits_b], axis=0)
    return (top_k_probs, top_k_indices, router_logits)


# speedup vs baseline: 1.8394x; 1.8394x over previous
import jax
import jax.numpy as jnp
from jax import lax
from jax.experimental import pallas as pl
from jax.experimental.pallas import tpu as pltpu
from jax.experimental.pallas import tpu_sc as plsc

_EPS = 1e-05
_BLK = 2048
_E = 16
_NW = 32
_L = 16


def _layernorm_logits(x, w):
    h = x.shape[1]
    s1 = jnp.sum(x, axis=1, keepdims=True)
    s2 = jnp.sum(x * x, axis=1, keepdims=True)
    mu = s1 / h
    var = s2 / h - mu * mu
    rstd = lax.rsqrt(var + 1e-5)
    hn = (x - mu) * rstd
    g = lax.dot_general(
        hn, w, (((1,), (1,)), ((), ())),
        preferred_element_type=jnp.float32)
    return jnp.clip(g, -20.0, 20.0)


def _logits_block(x_ref, w_ref, logits_ref):
    logits_ref[...] = _layernorm_logits(x_ref[...], w_ref[...])


def _router_block(x_ref, w_ref, p_ref, i_ref, logits_ref):
    logits = _layernorm_logits(x_ref[...], w_ref[...])
    logits_ref[...] = logits
    bb, ee = logits.shape
    m = jnp.max(logits, axis=1, keepdims=True)
    ex = jnp.exp(logits - m)
    p = ex / jnp.sum(ex, axis=1, keepdims=True)
    p = jnp.clip(p, _EPS, 1.0)
    iota = lax.broadcasted_iota(jnp.int32, (bb, ee), 1)
    m1 = jnp.max(p, axis=1, keepdims=True)
    i1 = jnp.min(jnp.where(p == m1, iota, ee), axis=1, keepdims=True)
    masked = jnp.where(iota == i1, -1.0, p)
    m2 = jnp.max(masked, axis=1, keepdims=True)
    i2 = jnp.min(jnp.where(masked == m2, iota, ee), axis=1, keepdims=True)
    ps = jnp.maximum(m1 + m2, _EPS)
    p_ref[:, 0:1] = m1 / ps
    p_ref[:, 1:2] = m2 / ps
    i_ref[:, 0:1] = i1
    i_ref[:, 1:2] = i2


def _route_sc(logits_hbm, p1_hbm, p2_hbm, i1_hbm, i2_hbm,
              chunk_v, p1_v, p2_v, i1_v, i2_v):
    c = p1_v.shape[0]
    wid = lax.axis_index("s") * 2 + lax.axis_index("c")
    base = wid * c
    pltpu.sync_copy(logits_hbm.at[pl.ds(base, c), :], chunk_v)
    lane = lax.iota(jnp.int32, _L)
    for g in range(c // _L):
        row = lane + g * _L
        cols = [plsc.load_gather(chunk_v, [row, jnp.full((_L,), e, jnp.int32)])
                for e in range(_E)]
        m = cols[0]
        for e in range(1, _E):
            m = jnp.maximum(m, cols[e])
        exps = [jnp.exp(x - m) for x in cols]
        s = exps[0]
        for e in range(1, _E):
            s = s + exps[e]
        rinv = 1.0 / s
        v1 = jnp.clip(exps[0] * rinv, _EPS, 1.0)
        i1 = jnp.zeros((_L,), jnp.int32)
        v2 = jnp.full((_L,), -1.0, jnp.float32)
        i2 = jnp.zeros((_L,), jnp.int32)
        for e in range(1, _E):
            p = jnp.clip(exps[e] * rinv, _EPS, 1.0)
            gt1 = p > v1
            gt2 = p > v2
            v2 = jnp.where(gt1, v1, jnp.where(gt2, p, v2))
            i2 = jnp.where(gt1, i1, jnp.where(gt2, jnp.full((_L,), e, jnp.int32), i2))
            v1 = jnp.where(gt1, p, v1)
            i1 = jnp.where(gt1, jnp.full((_L,), e, jnp.int32), i1)
        ps = jnp.maximum(v1 + v2, _EPS)
        rs = 1.0 / ps
        sl = pl.ds(g * _L, _L)
        p1_v[sl] = v1 * rs
        p2_v[sl] = v2 * rs
        i1_v[sl] = i1
        i2_v[sl] = i2
    out_sl = pl.ds(base, c)
    pltpu.sync_copy(p1_v, p1_hbm.at[out_sl])
    pltpu.sync_copy(p2_v, p2_hbm.at[out_sl])
    pltpu.sync_copy(i1_v, i1_hbm.at[out_sl])
    pltpu.sync_copy(i2_v, i2_hbm.at[out_sl])


def kernel(hidden_states, gate_weight):
    b, s, h = hidden_states.shape
    e = gate_weight.shape[0]
    n = b * s
    na = n // 2
    nb = n - na
    c = na // _NW
    x = hidden_states.reshape(n, h)

    logits_a = pl.pallas_call(
        _logits_block,
        grid=(na // _BLK,),
        in_specs=[
            pl.BlockSpec((_BLK, h), lambda i: (i, 0)),
            pl.BlockSpec((e, h), lambda i: (0, 0)),
        ],
        out_specs=pl.BlockSpec((_BLK, e), lambda i: (i, 0)),
        out_shape=jax.ShapeDtypeStruct((na, e), jnp.float32),
        compiler_params=pltpu.CompilerParams(
            dimension_semantics=("arbitrary",)),
    )(x, gate_weight)

    route = pl.kernel(
        _route_sc,
        out_type=[
            jax.ShapeDtypeStruct((na,), jnp.float32),
            jax.ShapeDtypeStruct((na,), jnp.float32),
            jax.ShapeDtypeStruct((na,), jnp.int32),
            jax.ShapeDtypeStruct((na,), jnp.int32),
        ],
        mesh=plsc.VectorSubcoreMesh(core_axis_name="c", subcore_axis_name="s"),
        compiler_params=pltpu.CompilerParams(needs_layout_passes=False),
        scratch_types=[
            pltpu.VMEM((c, _E), jnp.float32),
            pltpu.VMEM((c,), jnp.float32),
            pltpu.VMEM((c,), jnp.float32),
            pltpu.VMEM((c,), jnp.int32),
            pltpu.VMEM((c,), jnp.int32),
        ],
    )
    pa1, pa2, ia1, ia2 = route(logits_a)

    nblk_a = na // _BLK
    pb, ib, logits_b = pl.pallas_call(
        _router_block,
        grid=(nb // _BLK,),
        in_specs=[
            pl.BlockSpec((_BLK, h), lambda i: (i + nblk_a, 0)),
            pl.BlockSpec((e, h), lambda i: (0, 0)),
        ],
        out_specs=[
            pl.BlockSpec((_BLK, 2), lambda i: (i, 0)),
            pl.BlockSpec((_BLK, 2), lambda i: (i, 0)),
            pl.BlockSpec((_BLK, e), lambda i: (i, 0)),
        ],
        out_shape=[
            jax.ShapeDtypeStruct((nb, 2), jnp.float32),
            jax.ShapeDtypeStruct((nb, 2), jnp.int32),
            jax.ShapeDtypeStruct((nb, e), jnp.float32),
        ],
        compiler_params=pltpu.CompilerParams(
            dimension_semantics=("arbitrary",)),
    )(x, gate_weight)

    top_k_probs = jnp.concatenate(
        [jnp.stack([pa1, pa2], axis=-1), pb], axis=0)
    top_k_indices = jnp.concatenate(
        [jnp.stack([ia1, ia2], axis=-1), ib], axis=0)
    router_logits = jnp.concatenate([logits_a, logits_b], axis=0)
    return (top_k_probs, top_k_indices, router_logits)


# speedup vs baseline: 1.9418x; 1.0557x over previous
import jax
import jax.numpy as jnp
from jax import lax
from jax.experimental import pallas as pl
from jax.experimental.pallas import tpu as pltpu
from jax.experimental.pallas import tpu_sc as plsc

_EPS = 1e-05
_BLK = 1024
_E = 16
_NW = 32
_L = 16


def _layernorm_logits(x, w):
    h = x.shape[1]
    s1 = jnp.sum(x, axis=1, keepdims=True)
    s2 = jnp.sum(x * x, axis=1, keepdims=True)
    mu = s1 / h
    var = s2 / h - mu * mu
    rstd = lax.rsqrt(var + 1e-5)
    hn = (x - mu) * rstd
    g = lax.dot_general(
        hn, w, (((1,), (1,)), ((), ())),
        preferred_element_type=jnp.float32)
    return jnp.clip(g, -20.0, 20.0)


def _logits_block(x_ref, w_ref, logits_ref):
    logits_ref[...] = _layernorm_logits(x_ref[...], w_ref[...])


def _router_block(x_ref, w_ref, p_ref, i_ref, logits_ref):
    logits = _layernorm_logits(x_ref[...], w_ref[...])
    logits_ref[...] = logits
    bb, ee = logits.shape
    m = jnp.max(logits, axis=1, keepdims=True)
    ex = jnp.exp(logits - m)
    p = ex / jnp.sum(ex, axis=1, keepdims=True)
    p = jnp.clip(p, _EPS, 1.0)
    iota = lax.broadcasted_iota(jnp.int32, (bb, ee), 1)
    m1 = jnp.max(p, axis=1, keepdims=True)
    i1 = jnp.min(jnp.where(p == m1, iota, ee), axis=1, keepdims=True)
    masked = jnp.where(iota == i1, -1.0, p)
    m2 = jnp.max(masked, axis=1, keepdims=True)
    i2 = jnp.min(jnp.where(masked == m2, iota, ee), axis=1, keepdims=True)
    ps = jnp.maximum(m1 + m2, _EPS)
    p_ref[:, 0:1] = m1 / ps
    p_ref[:, 1:2] = m2 / ps
    i_ref[:, 0:1] = i1
    i_ref[:, 1:2] = i2


def _route_sc(logits_hbm, p1_hbm, p2_hbm, i1_hbm, i2_hbm,
              chunk_v, p1_v, p2_v, i1_v, i2_v):
    c = p1_v.shape[0]
    wid = lax.axis_index("s") * 2 + lax.axis_index("c")
    base = wid * c
    pltpu.sync_copy(logits_hbm.at[pl.ds(base, c), :], chunk_v)
    lane = lax.iota(jnp.int32, _L)
    for g in range(c // _L):
        row = lane + g * _L
        cols = [plsc.load_gather(chunk_v, [row, jnp.full((_L,), e, jnp.int32)])
                for e in range(_E)]
        m = cols[0]
        for e in range(1, _E):
            m = jnp.maximum(m, cols[e])
        exps = [jnp.exp(x - m) for x in cols]
        s = exps[0]
        for e in range(1, _E):
            s = s + exps[e]
        rinv = 1.0 / s
        v1 = jnp.clip(exps[0] * rinv, _EPS, 1.0)
        i1 = jnp.zeros((_L,), jnp.int32)
        v2 = jnp.full((_L,), -1.0, jnp.float32)
        i2 = jnp.zeros((_L,), jnp.int32)
        for e in range(1, _E):
            p = jnp.clip(exps[e] * rinv, _EPS, 1.0)
            gt1 = p > v1
            gt2 = p > v2
            v2 = jnp.where(gt1, v1, jnp.where(gt2, p, v2))
            i2 = jnp.where(gt1, i1, jnp.where(gt2, jnp.full((_L,), e, jnp.int32), i2))
            v1 = jnp.where(gt1, p, v1)
            i1 = jnp.where(gt1, jnp.full((_L,), e, jnp.int32), i1)
        ps = jnp.maximum(v1 + v2, _EPS)
        rs = 1.0 / ps
        sl = pl.ds(g * _L, _L)
        p1_v[sl] = v1 * rs
        p2_v[sl] = v2 * rs
        i1_v[sl] = i1
        i2_v[sl] = i2
    out_sl = pl.ds(base, c)
    pltpu.sync_copy(p1_v, p1_hbm.at[out_sl])
    pltpu.sync_copy(p2_v, p2_hbm.at[out_sl])
    pltpu.sync_copy(i1_v, i1_hbm.at[out_sl])
    pltpu.sync_copy(i2_v, i2_hbm.at[out_sl])


def kernel(hidden_states, gate_weight):
    b, s, h = hidden_states.shape
    e = gate_weight.shape[0]
    n = b * s
    na = n // 2
    nb = n - na
    c = na // _NW
    x = hidden_states.reshape(n, h)

    logits_a = pl.pallas_call(
        _logits_block,
        grid=(na // _BLK,),
        in_specs=[
            pl.BlockSpec((_BLK, h), lambda i: (i, 0)),
            pl.BlockSpec((e, h), lambda i: (0, 0)),
        ],
        out_specs=pl.BlockSpec((_BLK, e), lambda i: (i, 0)),
        out_shape=jax.ShapeDtypeStruct((na, e), jnp.float32),
        compiler_params=pltpu.CompilerParams(
            dimension_semantics=("arbitrary",)),
    )(x, gate_weight)

    route = pl.kernel(
        _route_sc,
        out_type=[
            jax.ShapeDtypeStruct((na,), jnp.float32),
            jax.ShapeDtypeStruct((na,), jnp.float32),
            jax.ShapeDtypeStruct((na,), jnp.int32),
            jax.ShapeDtypeStruct((na,), jnp.int32),
        ],
        mesh=plsc.VectorSubcoreMesh(core_axis_name="c", subcore_axis_name="s"),
        compiler_params=pltpu.CompilerParams(needs_layout_passes=False),
        scratch_types=[
            pltpu.VMEM((c, _E), jnp.float32),
            pltpu.VMEM((c,), jnp.float32),
            pltpu.VMEM((c,), jnp.float32),
            pltpu.VMEM((c,), jnp.int32),
            pltpu.VMEM((c,), jnp.int32),
        ],
    )
    pa1, pa2, ia1, ia2 = route(logits_a)

    nblk_a = na // _BLK
    pb, ib, logits_b = pl.pallas_call(
        _router_block,
        grid=(nb // _BLK,),
        in_specs=[
            pl.BlockSpec((_BLK, h), lambda i: (i + nblk_a, 0)),
            pl.BlockSpec((e, h), lambda i: (0, 0)),
        ],
        out_specs=[
            pl.BlockSpec((_BLK, 2), lambda i: (i, 0)),
            pl.BlockSpec((_BLK, 2), lambda i: (i, 0)),
            pl.BlockSpec((_BLK, e), lambda i: (i, 0)),
        ],
        out_shape=[
            jax.ShapeDtypeStruct((nb, 2), jnp.float32),
            jax.ShapeDtypeStruct((nb, 2), jnp.int32),
            jax.ShapeDtypeStruct((nb, e), jnp.float32),
        ],
        compiler_params=pltpu.CompilerParams(
            dimension_semantics=("arbitrary",)),
    )(x, gate_weight)

    top_k_probs = jnp.concatenate(
        [jnp.stack([pa1, pa2], axis=-1), pb], axis=0)
    top_k_indices = jnp.concatenate(
        [jnp.stack([ia1, ia2], axis=-1), ib], axis=0)
    router_logits = jnp.concatenate([logits_a, logits_b], axis=0)
    return (top_k_probs, top_k_indices, router_logits)
